# Initial kernel scaffold; baseline (speedup 1.0000x reference)
import jax
import jax.numpy as jnp
from jax.experimental import pallas as pl


def kernel(X, edge_index, edge_weight, p, W_conv, W_ih, W_hh, b_ih, b_hh):
    raise NotImplementedError("write your pallas kernel here")


# speedup vs baseline: 14.1069x; 14.1069x over previous
import functools

import jax
import jax.numpy as jnp
from jax import lax
from jax.experimental import pallas as pl
from jax.experimental.pallas import tpu as pltpu
from jax.experimental.pallas import tpu_sc as plsc

N = 10000
D = 128
E = 320000
K = 128

NC = 2
NS = 16
NW = NC * NS
EPW = E // NW
CH = 80
NCHUNK = EPW // CH
RB = 624
ZCH = 208

_SC_PARAMS = pltpu.CompilerParams(needs_layout_passes=False)


@functools.lru_cache(maxsize=None)
def _mesh():
    return plsc.VectorSubcoreMesh(core_axis_name="c", subcore_axis_name="s")


def _p1_body(dst_hbm, w_hbm, deg_hbm, deg_v, dst_v, w_v):
    c = lax.axis_index("c")
    s = lax.axis_index("s")
    wid = s * NC + c

    @pl.loop(0, N // 16)
    def _(i):
        for cc in range(0, 16, 16):
            deg_v[i, pl.ds(cc, 16)] = jnp.zeros((16,), jnp.float32)

    ebase = wid * EPW
    CH1 = 2000

    @pl.loop(0, EPW, step=CH1)
    def _(eoff):
        pltpu.sync_copy(dst_hbm.at[pl.ds(ebase + eoff, CH1)], dst_v)
        pltpu.sync_copy(w_hbm.at[pl.ds(ebase + eoff, CH1)], w_v)

        @pl.loop(0, CH1, step=16)
        def _(j):
            d16 = dst_v[pl.ds(j, 16)]
            w16 = w_v[pl.ds(j, 16)]
            row = lax.shift_right_logical(d16, 4)
            col = lax.bitwise_and(d16, 15)
            plsc.addupdate_scatter(deg_v, [row, col], w16)

    pltpu.sync_copy(deg_v, deg_hbm.at[wid])


@functools.lru_cache(maxsize=None)
def _p1():
    return pl.kernel(
        _p1_body,
        out_type=jax.ShapeDtypeStruct((NW, N // 16, 16), jnp.float32),
        mesh=_mesh(),
        scratch_types=[
            pltpu.VMEM((N // 16, 16), jnp.float32),
            pltpu.VMEM((2000,), jnp.int32),
            pltpu.VMEM((2000,), jnp.float32),
        ],
        compiler_params=_SC_PARAMS,
    )


def _dinv_body(parts_ref, dinv_ref):
    dinv_ref[...] = lax.rsqrt(jnp.sum(parts_ref[...], axis=0) + 1.0)


_dinv = pl.pallas_call(
    _dinv_body,
    out_shape=jax.ShapeDtypeStruct((N,), jnp.float32),
)


def _p2_body(x_hbm, src_hbm, dst_hbm, w_hbm, dinv_hbm, agg_hbm,
             src_v, dst_v, w_v, s_v, rows_v, dinv_v, zero_v, acc_sh):
    c = lax.axis_index("c")
    s = lax.axis_index("s")
    wid = s * NC + c

    pltpu.sync_copy(dinv_hbm, dinv_v)

    @pl.loop(0, ZCH)
    def _(r):
        for cc in range(0, D, 16):
            zero_v[r, pl.ds(cc, 16)] = jnp.zeros((16,), jnp.float32)

    rbase = pl.multiple_of(s * RB, 8)

    @pl.loop(0, RB, step=ZCH)
    def _(r0):
        pltpu.sync_copy(zero_v, acc_sh.at[pl.ds(rbase + r0, ZCH)])

    @pl.when(s == NS - 1)
    def _():
        pltpu.sync_copy(zero_v.at[pl.ds(0, 16)], acc_sh.at[pl.ds(NS * RB, 16)])

    plsc.subcore_barrier()

    ebase = wid * EPW

    @pl.loop(0, NCHUNK)
    def _(i):
        off = ebase + i * CH
        pltpu.sync_copy(src_hbm.at[pl.ds(off, CH)], src_v)
        pltpu.sync_copy(dst_hbm.at[pl.ds(off, CH)], dst_v)
        pltpu.sync_copy(w_hbm.at[pl.ds(off, CH)], w_v)
        pltpu.sync_copy(x_hbm.at[src_v], rows_v)

        @pl.loop(0, CH, step=16)
        def _(j):
            idx16 = src_v[pl.ds(j, 16)]
            dg = plsc.load_gather(dinv_v, [idx16])
            s_v[pl.ds(j, 16)] = w_v[pl.ds(j, 16)] * dg

        @pl.loop(0, CH)
        def _(j):
            jvec = jnp.full((16,), 0, jnp.int32) + j
            sj = plsc.load_gather(s_v, [jvec])
            for cc in range(0, D, 16):
                rows_v[j, pl.ds(cc, 16)] = rows_v[j, pl.ds(cc, 16)] * sj

        pltpu.sync_copy(rows_v, acc_sh.at[dst_v], add=True)

    plsc.subcore_barrier()

    @pl.loop(0, RB, step=ZCH)
    def _(r0):
        pltpu.sync_copy(acc_sh.at[pl.ds(rbase + r0, ZCH)],
                        agg_hbm.at[c, pl.ds(rbase + r0, ZCH)])

    @pl.when(s == NS - 1)
    def _():
        pltpu.sync_copy(acc_sh.at[pl.ds(NS * RB, 16)],
                        agg_hbm.at[c, pl.ds(NS * RB, 16)])


@functools.lru_cache(maxsize=None)
def _p2():
    return pl.kernel(
        _p2_body,
        out_type=jax.ShapeDtypeStruct((NC, N, D), jnp.float32),
        mesh=_mesh(),
        scratch_types=[
            pltpu.VMEM((CH,), jnp.int32),
            pltpu.VMEM((CH,), jnp.int32),
            pltpu.VMEM((CH,), jnp.float32),
            pltpu.VMEM((CH,), jnp.float32),
            pltpu.VMEM((CH, D), jnp.float32),
            pltpu.VMEM((N,), jnp.float32),
            pltpu.VMEM((ZCH, D), jnp.float32),
            pltpu.VMEM_SHARED((N, D), jnp.float32),
        ],
        compiler_params=_SC_PARAMS,
    )


def _wnew_body(x_ref, p_ref, wc_ref, wih_ref, whh_ref, bih_ref, bhh_ref,
               wnew_ref, s_scr, xt_scr, idx_s, topv_s):
    p = p_ref[...]
    pn = jnp.sqrt(jnp.sum(p * p))
    s_scr[...] = lax.dot_general(
        x_ref[...].astype(jnp.bfloat16), p.astype(jnp.bfloat16),
        (((1,), (0,)), ((), ())),
        preferred_element_type=jnp.float32) / pn

    lin = lax.broadcasted_iota(jnp.int32, (N,), 0)

    def sel_body(k, carry):
        sv = s_scr[...]
        m = jnp.max(sv)
        idx = jnp.min(jnp.where(sv >= m, lin, jnp.int32(N)))
        idx_s[k] = idx
        topv_s[k] = m
        s_scr[...] = jnp.where(lin == idx, -jnp.inf, sv)
        return carry

    lax.fori_loop(0, K, sel_body, 0)

    def gat_body(k, carry):
        i = idx_s[k]
        g = jnp.tanh(topv_s[k])
        xt_scr[pl.ds(k, 1), :] = x_ref[pl.ds(i, 1), :] * g
        return carry

    lax.fori_loop(0, K, gat_body, 0)

    xt = xt_scr[...]
    h = wc_ref[...]
    dn = (((1,), (1,)), ((), ()))
    gi = lax.dot_general(xt, wih_ref[...], dn, precision=lax.Precision.HIGHEST,
                         preferred_element_type=jnp.float32) + bih_ref[...][None, :]
    gh = lax.dot_general(h, whh_ref[...], dn, precision=lax.Precision.HIGHEST,
                         preferred_element_type=jnp.float32) + bhh_ref[...][None, :]
    r = jax.nn.sigmoid(gi[:, 0:D] + gh[:, 0:D])
    z = jax.nn.sigmoid(gi[:, D:2 * D] + gh[:, D:2 * D])
    n = jnp.tanh(gi[:, 2 * D:3 * D] + r * gh[:, 2 * D:3 * D])
    wnew_ref[...] = (1.0 - z) * n + z * h


_wnew = pl.pallas_call(
    _wnew_body,
    out_shape=jax.ShapeDtypeStruct((D, D), jnp.float32),
    scratch_shapes=[
        pltpu.VMEM((N,), jnp.float32),
        pltpu.VMEM((K, D), jnp.float32),
        pltpu.SMEM((K,), jnp.int32),
        pltpu.SMEM((K,), jnp.float32),
    ],
)


BR = 2000


def _out_body(x_ref, a0_ref, a1_ref, dinv_ref, wn_ref, o_ref):
    dcol = dinv_ref[...]
    t = a0_ref[...] + a1_ref[...] + dcol * x_ref[...]
    t = dcol * t
    o_ref[...] = lax.dot_general(t, wn_ref[...], (((1,), (1,)), ((), ())),
                                 precision=lax.Precision.HIGHEST,
                                 preferred_element_type=jnp.float32)


_outk = pl.pallas_call(
    _out_body,
    grid=(N // BR,),
    in_specs=[
        pl.BlockSpec((BR, D), lambda i: (i, 0)),
        pl.BlockSpec((BR, D), lambda i: (i, 0)),
        pl.BlockSpec((BR, D), lambda i: (i, 0)),
        pl.BlockSpec((BR, 1), lambda i: (i, 0)),
        pl.BlockSpec((D, D), lambda i: (0, 0)),
    ],
    out_specs=pl.BlockSpec((BR, D), lambda i: (i, 0)),
    out_shape=jax.ShapeDtypeStruct((N, D), jnp.float32),
)


def kernel(X, edge_index, edge_weight, p, W_conv, W_ih, W_hh, b_ih, b_hh):
    src = edge_index[0]
    dst = edge_index[1]
    deg_parts = _p1()(dst, edge_weight).reshape(NW, N)
    dinv = _dinv(deg_parts)
    agg = _p2()(X, src, dst, edge_weight, dinv)
    w_new = _wnew(X, p, W_conv, W_ih, W_hh, b_ih, b_hh)
    out = _outk(X, agg[0], agg[1], dinv.reshape(N, 1), w_new)
    return out


# speedup vs baseline: 22.2402x; 1.5765x over previous
import functools

import jax
import jax.numpy as jnp
from jax import lax
from jax.experimental import pallas as pl
from jax.experimental.pallas import tpu as pltpu
from jax.experimental.pallas import tpu_sc as plsc

N = 10000
D = 128
E = 320000
K = 128

NC = 2
NS = 16
NW = NC * NS
EPW = E // NW
CH = 48
NCHUNK = 210
EPWP = NCHUNK * CH
NPAD = N + 8
RB = 624
ZCH = 48

_SC_PARAMS = pltpu.CompilerParams(needs_layout_passes=False)


@functools.lru_cache(maxsize=None)
def _mesh():
    return plsc.VectorSubcoreMesh(core_axis_name="c", subcore_axis_name="s")


def _p1_body(dst_hbm, w_hbm, deg_hbm, deg_v, dst_v, w_v):
    c = lax.axis_index("c")
    s = lax.axis_index("s")
    wid = s * NC + c

    @pl.loop(0, N // 16)
    def _(i):
        for cc in range(0, 16, 16):
            deg_v[i, pl.ds(cc, 16)] = jnp.zeros((16,), jnp.float32)

    ebase = wid * EPW
    CH1 = 2000

    @pl.loop(0, EPW, step=CH1)
    def _(eoff):
        pltpu.sync_copy(dst_hbm.at[pl.ds(ebase + eoff, CH1)], dst_v)
        pltpu.sync_copy(w_hbm.at[pl.ds(ebase + eoff, CH1)], w_v)

        @pl.loop(0, CH1, step=16)
        def _(j):
            d16 = dst_v[pl.ds(j, 16)]
            w16 = w_v[pl.ds(j, 16)]
            row = lax.shift_right_logical(d16, 4)
            col = lax.bitwise_and(d16, 15)
            plsc.addupdate_scatter(deg_v, [row, col], w16)

    pltpu.sync_copy(deg_v, deg_hbm.at[wid])


@functools.lru_cache(maxsize=None)
def _p1():
    return pl.kernel(
        _p1_body,
        out_type=jax.ShapeDtypeStruct((NW, N // 16, 16), jnp.float32),
        mesh=_mesh(),
        scratch_types=[
            pltpu.VMEM((N // 16, 16), jnp.float32),
            pltpu.VMEM((2000,), jnp.int32),
            pltpu.VMEM((2000,), jnp.float32),
        ],
        compiler_params=_SC_PARAMS,
    )


def _dinv_body(parts_ref, dinv_ref):
    dinv_ref[...] = lax.rsqrt(jnp.sum(parts_ref[...], axis=0) + 1.0)


_dinv = pl.pallas_call(
    _dinv_body,
    out_shape=jax.ShapeDtypeStruct((N,), jnp.float32),
)


def _p2_body(x_hbm, pk_hbm, w_hbm, dinv_hbm, agg_hbm,
             pk_v, w_v, s_v, src0, src1, src2, dst0, dst1, dst2,
             rows0, rows1, rows2, dinv_v, acc_sh, gsems, ssems):
    c = lax.axis_index("c")
    s = lax.axis_index("s")
    wid = s * NC + c
    rows = [rows0, rows1, rows2]
    srcb = [src0, src1, src2]
    dstb = [dst0, dst1, dst2]

    pltpu.sync_copy(pk_hbm.at[wid], pk_v)
    pltpu.sync_copy(w_hbm.at[wid], w_v)
    pltpu.sync_copy(dinv_hbm, dinv_v)

    def unpack_idx(ci, b):
        @pl.loop(0, CH, step=16)
        def _(j):
            pk16 = pk_v[pl.ds(ci * CH + j, 16)]
            srcb[b][pl.ds(j, 16)] = lax.bitwise_and(pk16, 0xFFFF)
            dstb[b][pl.ds(j, 16)] = lax.shift_right_logical(pk16, 16)

    def issue_gather(ci, b):
        unpack_idx(ci, b)
        pltpu.async_copy(x_hbm.at[srcb[b]], rows[b], gsems.at[b])

    def wait_gather(b):
        pltpu.make_async_copy(x_hbm.at[srcb[b]], rows[b], gsems.at[b]).wait()

    def issue_scatter(b):
        pltpu.async_copy(rows[b], acc_sh.at[dstb[b]], ssems.at[b], add=True)

    def wait_scatter(b):
        pltpu.make_async_copy(rows[b], acc_sh.at[dstb[b]],
                              ssems.at[b]).wait()

    issue_gather(0, 0)
    issue_gather(1, 1)

    @pl.loop(0, ZCH)
    def _(r):
        for cc in range(0, D, 16):
            rows2[r, pl.ds(cc, 16)] = jnp.zeros((16,), jnp.float32)

    rbase = pl.multiple_of(s * RB, 8)

    @pl.loop(0, RB, step=ZCH)
    def _(r0):
        pltpu.sync_copy(rows2, acc_sh.at[pl.ds(rbase + r0, ZCH)])

    @pl.when(s == NS - 1)
    def _():
        pltpu.sync_copy(rows2.at[pl.ds(0, NPAD - NS * RB)],
                        acc_sh.at[pl.ds(NS * RB, NPAD - NS * RB)])

    plsc.subcore_barrier()

    def process(ci, b):
        wait_gather(b)

        @pl.loop(0, CH, step=16)
        def _(j):
            idx16 = srcb[b][pl.ds(j, 16)]
            dg = plsc.load_gather(dinv_v, [idx16])
            s_v[pl.ds(j, 16)] = w_v[pl.ds(ci * CH + j, 16)] * dg

        rv = rows[b]

        @pl.loop(0, CH)
        def _(j):
            jvec = jnp.full((16,), 0, jnp.int32) + j
            sj = plsc.load_gather(s_v, [jvec])
            for cc in range(0, D, 16):
                rv[j, pl.ds(cc, 16)] = rv[j, pl.ds(cc, 16)] * sj

        issue_scatter(b)

        nxt = ci + 2

        @pl.when(nxt < NCHUNK)
        def _():
            nb = (b + 2) % 3

            @pl.when(nxt >= 3)
            def _():
                wait_scatter(nb)

            issue_gather(nxt, nb)

    @pl.loop(0, NCHUNK // 3)
    def _(ii):
        for u in range(3):
            process(ii * 3 + u, u)

    for b in (0, 1, 2):
        wait_scatter(b)

    plsc.subcore_barrier()

    @pl.loop(0, RB, step=ZCH)
    def _(r0):
        pltpu.sync_copy(acc_sh.at[pl.ds(rbase + r0, ZCH)],
                        agg_hbm.at[c, pl.ds(rbase + r0, ZCH)])

    @pl.when(s == NS - 1)
    def _():
        pltpu.sync_copy(acc_sh.at[pl.ds(NS * RB, 16)],
                        agg_hbm.at[c, pl.ds(NS * RB, 16)])


@functools.lru_cache(maxsize=None)
def _p2():
    return pl.kernel(
        _p2_body,
        out_type=jax.ShapeDtypeStruct((NC, N, D), jnp.float32),
        mesh=_mesh(),
        scratch_types=[
            pltpu.VMEM((EPWP,), jnp.int32),
            pltpu.VMEM((EPWP,), jnp.float32),
            pltpu.VMEM((CH,), jnp.float32),
            pltpu.VMEM((CH,), jnp.int32),
            pltpu.VMEM((CH,), jnp.int32),
            pltpu.VMEM((CH,), jnp.int32),
            pltpu.VMEM((CH,), jnp.int32),
            pltpu.VMEM((CH,), jnp.int32),
            pltpu.VMEM((CH,), jnp.int32),
            pltpu.VMEM((CH, D), jnp.float32),
            pltpu.VMEM((CH, D), jnp.float32),
            pltpu.VMEM((CH, D), jnp.float32),
            pltpu.VMEM((N,), jnp.float32),
            pltpu.VMEM_SHARED((NPAD, D), jnp.float32),
            pltpu.SemaphoreType.DMA((3,)),
            pltpu.SemaphoreType.DMA((3,)),
        ],
        compiler_params=_SC_PARAMS,
    )


def _wnew_body(x_ref, p_ref, wc_ref, wih_ref, whh_ref, bih_ref, bhh_ref,
               wnew_ref, s_scr, xt_scr, idx_s, topv_s):
    p = p_ref[...]
    pn = jnp.sqrt(jnp.sum(p * p))
    s_scr[...] = lax.dot_general(
        x_ref[...].astype(jnp.bfloat16), p.astype(jnp.bfloat16),
        (((1,), (0,)), ((), ())),
        preferred_element_type=jnp.float32) / pn

    lin = lax.broadcasted_iota(jnp.int32, (N,), 0)

    def sel_body(k, carry):
        sv = s_scr[...]
        m = jnp.max(sv)
        idx = jnp.min(jnp.where(sv >= m, lin, jnp.int32(N)))
        idx_s[k] = idx
        topv_s[k] = m
        s_scr[...] = jnp.where(lin == idx, -jnp.inf, sv)
        return carry

    lax.fori_loop(0, K, sel_body, 0)

    def gat_body(k, carry):
        i = idx_s[k]
        g = jnp.tanh(topv_s[k])
        xt_scr[pl.ds(k, 1), :] = x_ref[pl.ds(i, 1), :] * g
        return carry

    lax.fori_loop(0, K, gat_body, 0)

    xt = xt_scr[...]
    h = wc_ref[...]
    dn = (((1,), (1,)), ((), ()))
    gi = lax.dot_general(xt, wih_ref[...], dn, precision=lax.Precision.HIGHEST,
                         preferred_element_type=jnp.float32) + bih_ref[...][None, :]
    gh = lax.dot_general(h, whh_ref[...], dn, precision=lax.Precision.HIGHEST,
                         preferred_element_type=jnp.float32) + bhh_ref[...][None, :]
    r = jax.nn.sigmoid(gi[:, 0:D] + gh[:, 0:D])
    z = jax.nn.sigmoid(gi[:, D:2 * D] + gh[:, D:2 * D])
    n = jnp.tanh(gi[:, 2 * D:3 * D] + r * gh[:, 2 * D:3 * D])
    wnew_ref[...] = (1.0 - z) * n + z * h


_wnew = pl.pallas_call(
    _wnew_body,
    out_shape=jax.ShapeDtypeStruct((D, D), jnp.float32),
    scratch_shapes=[
        pltpu.VMEM((N,), jnp.float32),
        pltpu.VMEM((K, D), jnp.float32),
        pltpu.SMEM((K,), jnp.int32),
        pltpu.SMEM((K,), jnp.float32),
    ],
)


BR = 2000


def _out_body(x_ref, a0_ref, a1_ref, dinv_ref, wn_ref, o_ref):
    dcol = dinv_ref[...]
    t = a0_ref[...] + a1_ref[...] + dcol * x_ref[...]
    t = dcol * t
    o_ref[...] = lax.dot_general(t, wn_ref[...], (((1,), (1,)), ((), ())),
                                 precision=lax.Precision.HIGHEST,
                                 preferred_element_type=jnp.float32)


_outk = pl.pallas_call(
    _out_body,
    grid=(N // BR,),
    in_specs=[
        pl.BlockSpec((BR, D), lambda i: (i, 0)),
        pl.BlockSpec((BR, D), lambda i: (i, 0)),
        pl.BlockSpec((BR, D), lambda i: (i, 0)),
        pl.BlockSpec((BR, 1), lambda i: (i, 0)),
        pl.BlockSpec((D, D), lambda i: (0, 0)),
    ],
    out_specs=pl.BlockSpec((BR, D), lambda i: (i, 0)),
    out_shape=jax.ShapeDtypeStruct((N, D), jnp.float32),
)


def kernel(X, edge_index, edge_weight, p, W_conv, W_ih, W_hh, b_ih, b_hh):
    src = edge_index[0]
    dst = edge_index[1]
    deg_parts = _p1()(dst, edge_weight).reshape(NW, N)
    dinv = _dinv(deg_parts)
    pad = EPWP - EPW
    srcw = jnp.pad(src.reshape(NW, EPW), ((0, 0), (0, pad)))
    dstw = jnp.pad(dst.reshape(NW, EPW), ((0, 0), (0, pad)),
                   constant_values=N)
    ww = jnp.pad(edge_weight.reshape(NW, EPW), ((0, 0), (0, pad)))
    packed = srcw | (dstw << 16)
    agg = _p2()(X, packed, ww, dinv)
    w_new = _wnew(X, p, W_conv, W_ih, W_hh, b_ih, b_hh)
    out = _outk(X, agg[0], agg[1], dinv.reshape(N, 1), w_new)
    return out


# speedup vs baseline: 23.6770x; 1.0646x over previous
import functools

import jax
import jax.numpy as jnp
from jax import lax
from jax.experimental import pallas as pl
from jax.experimental.pallas import tpu as pltpu
from jax.experimental.pallas import tpu_sc as plsc

N = 10000
D = 128
E = 320000
K = 128

NC = 2
NS = 16
NW = NC * NS
EPW = E // NW
CH = 48
NCHUNK = 210
EPWP = NCHUNK * CH
NPAD = N + 8
RB = 624
ZCH = 48

_SC_PARAMS = pltpu.CompilerParams(needs_layout_passes=False)


@functools.lru_cache(maxsize=None)
def _mesh():
    return plsc.VectorSubcoreMesh(core_axis_name="c", subcore_axis_name="s")


def _p1_body(dst_hbm, w_hbm, deg_hbm, deg_v, dst_v, w_v):
    c = lax.axis_index("c")
    s = lax.axis_index("s")
    wid = s * NC + c

    @pl.loop(0, N // 16)
    def _(i):
        for cc in range(0, 16, 16):
            deg_v[i, pl.ds(cc, 16)] = jnp.zeros((16,), jnp.float32)

    ebase = wid * EPW
    CH1 = 2000

    @pl.loop(0, EPW, step=CH1)
    def _(eoff):
        pltpu.sync_copy(dst_hbm.at[pl.ds(ebase + eoff, CH1)], dst_v)
        pltpu.sync_copy(w_hbm.at[pl.ds(ebase + eoff, CH1)], w_v)

        @pl.loop(0, CH1, step=16)
        def _(j):
            d16 = dst_v[pl.ds(j, 16)]
            w16 = w_v[pl.ds(j, 16)]
            row = lax.shift_right_logical(d16, 4)
            col = lax.bitwise_and(d16, 15)
            plsc.addupdate_scatter(deg_v, [row, col], w16)

    pltpu.sync_copy(deg_v, deg_hbm.at[wid])


@functools.lru_cache(maxsize=None)
def _p1():
    return pl.kernel(
        _p1_body,
        out_type=jax.ShapeDtypeStruct((NW, N // 16, 16), jnp.float32),
        mesh=_mesh(),
        scratch_types=[
            pltpu.VMEM((N // 16, 16), jnp.float32),
            pltpu.VMEM((2000,), jnp.int32),
            pltpu.VMEM((2000,), jnp.float32),
        ],
        compiler_params=_SC_PARAMS,
    )


def _dinv_body(parts_ref, dinv_ref):
    dinv_ref[...] = lax.rsqrt(jnp.sum(parts_ref[...], axis=0) + 1.0)


_dinv = pl.pallas_call(
    _dinv_body,
    out_shape=jax.ShapeDtypeStruct((N,), jnp.float32),
)


def _p2_body(x_hbm, pk_hbm, w_hbm, dinv_hbm, agg_hbm,
             pk_v, w_v, s_v, src0, src1, src2, dst0, dst1, dst2,
             rows0, rows1, rows2, dinv_v, acc_sh, gsems, ssems):
    c = lax.axis_index("c")
    s = lax.axis_index("s")
    wid = s * NC + c
    rows = [rows0, rows1, rows2]
    srcb = [src0, src1, src2]
    dstb = [dst0, dst1, dst2]

    pltpu.sync_copy(pk_hbm.at[wid], pk_v)
    pltpu.sync_copy(w_hbm.at[wid], w_v)
    pltpu.sync_copy(dinv_hbm, dinv_v)

    def unpack_idx(ci, b):
        for j in range(0, CH, 16):
            pk16 = pk_v[pl.ds(ci * CH + j, 16)]
            srcb[b][pl.ds(j, 16)] = lax.bitwise_and(pk16, 0xFFFF)
            dstb[b][pl.ds(j, 16)] = lax.shift_right_logical(pk16, 16)

    def issue_gather(ci, b):
        unpack_idx(ci, b)
        pltpu.async_copy(x_hbm.at[srcb[b]], rows[b], gsems.at[b])

    def wait_gather(b):
        pltpu.make_async_copy(x_hbm.at[srcb[b]], rows[b], gsems.at[b]).wait()

    def issue_scatter(b):
        pltpu.async_copy(rows[b], acc_sh.at[dstb[b]], ssems.at[b], add=True)

    def wait_scatter(b):
        pltpu.make_async_copy(rows[b], acc_sh.at[dstb[b]],
                              ssems.at[b]).wait()

    issue_gather(0, 0)
    issue_gather(1, 1)

    @pl.loop(0, ZCH)
    def _(r):
        for cc in range(0, D, 16):
            rows2[r, pl.ds(cc, 16)] = jnp.zeros((16,), jnp.float32)

    rbase = pl.multiple_of(s * RB, 8)

    @pl.loop(0, RB, step=ZCH)
    def _(r0):
        pltpu.sync_copy(rows2, acc_sh.at[pl.ds(rbase + r0, ZCH)])

    @pl.when(s == NS - 1)
    def _():
        pltpu.sync_copy(rows2.at[pl.ds(0, NPAD - NS * RB)],
                        acc_sh.at[pl.ds(NS * RB, NPAD - NS * RB)])

    plsc.subcore_barrier()

    def process(ci, b):
        wait_gather(b)

        for j in range(0, CH, 16):
            idx16 = srcb[b][pl.ds(j, 16)]
            dg = plsc.load_gather(dinv_v, [idx16])
            s_v[pl.ds(j, 16)] = w_v[pl.ds(ci * CH + j, 16)] * dg

        rv = rows[b]

        @plsc.parallel_loop(0, CH, unroll=8)
        def _(j):
            jvec = jnp.full((16,), 0, jnp.int32) + j
            sj = plsc.load_gather(s_v, [jvec])
            for cc in range(0, D, 16):
                rv[j, pl.ds(cc, 16)] = rv[j, pl.ds(cc, 16)] * sj

        issue_scatter(b)

        nxt = ci + 2

        @pl.when(nxt < NCHUNK)
        def _():
            nb = (b + 2) % 3

            @pl.when(nxt >= 3)
            def _():
                wait_scatter(nb)

            issue_gather(nxt, nb)

    @pl.loop(0, NCHUNK // 3)
    def _(ii):
        for u in range(3):
            process(ii * 3 + u, u)

    for b in (0, 1, 2):
        wait_scatter(b)

    plsc.subcore_barrier()

    @pl.loop(0, RB, step=ZCH)
    def _(r0):
        pltpu.sync_copy(acc_sh.at[pl.ds(rbase + r0, ZCH)],
                        agg_hbm.at[c, pl.ds(rbase + r0, ZCH)])

    @pl.when(s == NS - 1)
    def _():
        pltpu.sync_copy(acc_sh.at[pl.ds(NS * RB, 16)],
                        agg_hbm.at[c, pl.ds(NS * RB, 16)])


@functools.lru_cache(maxsize=None)
def _p2():
    return pl.kernel(
        _p2_body,
        out_type=jax.ShapeDtypeStruct((NC, N, D), jnp.float32),
        mesh=_mesh(),
        scratch_types=[
            pltpu.VMEM((EPWP,), jnp.int32),
            pltpu.VMEM((EPWP,), jnp.float32),
            pltpu.VMEM((CH,), jnp.float32),
            pltpu.VMEM((CH,), jnp.int32),
            pltpu.VMEM((CH,), jnp.int32),
            pltpu.VMEM((CH,), jnp.int32),
            pltpu.VMEM((CH,), jnp.int32),
            pltpu.VMEM((CH,), jnp.int32),
            pltpu.VMEM((CH,), jnp.int32),
            pltpu.VMEM((CH, D), jnp.float32),
            pltpu.VMEM((CH, D), jnp.float32),
            pltpu.VMEM((CH, D), jnp.float32),
            pltpu.VMEM((N,), jnp.float32),
            pltpu.VMEM_SHARED((NPAD, D), jnp.float32),
            pltpu.SemaphoreType.DMA((3,)),
            pltpu.SemaphoreType.DMA((3,)),
        ],
        compiler_params=_SC_PARAMS,
    )


def _wnew_body(x_ref, p_ref, wc_ref, wih_ref, whh_ref, bih_ref, bhh_ref,
               wnew_ref, s_scr, xt_scr, idx_s, topv_s):
    p = p_ref[...]
    pn = jnp.sqrt(jnp.sum(p * p))
    s_scr[...] = lax.dot_general(
        x_ref[...].astype(jnp.bfloat16), p.astype(jnp.bfloat16),
        (((1,), (0,)), ((), ())),
        preferred_element_type=jnp.float32) / pn

    lin = lax.broadcasted_iota(jnp.int32, (N,), 0)

    def sel_body(k, carry):
        sv = s_scr[...]
        m = jnp.max(sv)
        idx = jnp.min(jnp.where(sv >= m, lin, jnp.int32(N)))
        idx_s[k] = idx
        topv_s[k] = m
        s_scr[...] = jnp.where(lin == idx, -jnp.inf, sv)
        return carry

    lax.fori_loop(0, K, sel_body, 0)

    def gat_body(k, carry):
        i = idx_s[k]
        g = jnp.tanh(topv_s[k])
        xt_scr[pl.ds(k, 1), :] = x_ref[pl.ds(i, 1), :] * g
        return carry

    lax.fori_loop(0, K, gat_body, 0)

    xt = xt_scr[...]
    h = wc_ref[...]
    dn = (((1,), (1,)), ((), ()))
    gi = lax.dot_general(xt, wih_ref[...], dn, precision=lax.Precision.HIGHEST,
                         preferred_element_type=jnp.float32) + bih_ref[...][None, :]
    gh = lax.dot_general(h, whh_ref[...], dn, precision=lax.Precision.HIGHEST,
                         preferred_element_type=jnp.float32) + bhh_ref[...][None, :]
    r = jax.nn.sigmoid(gi[:, 0:D] + gh[:, 0:D])
    z = jax.nn.sigmoid(gi[:, D:2 * D] + gh[:, D:2 * D])
    n = jnp.tanh(gi[:, 2 * D:3 * D] + r * gh[:, 2 * D:3 * D])
    wnew_ref[...] = (1.0 - z) * n + z * h


_wnew = pl.pallas_call(
    _wnew_body,
    out_shape=jax.ShapeDtypeStruct((D, D), jnp.float32),
    scratch_shapes=[
        pltpu.VMEM((N,), jnp.float32),
        pltpu.VMEM((K, D), jnp.float32),
        pltpu.SMEM((K,), jnp.int32),
        pltpu.SMEM((K,), jnp.float32),
    ],
)


BR = 2000


def _out_body(x_ref, a0_ref, a1_ref, dinv_ref, wn_ref, o_ref):
    dcol = dinv_ref[...]
    t = a0_ref[...] + a1_ref[...] + dcol * x_ref[...]
    t = dcol * t
    o_ref[...] = lax.dot_general(t, wn_ref[...], (((1,), (1,)), ((), ())),
                                 precision=lax.Precision.HIGHEST,
                                 preferred_element_type=jnp.float32)


_outk = pl.pallas_call(
    _out_body,
    grid=(N // BR,),
    in_specs=[
        pl.BlockSpec((BR, D), lambda i: (i, 0)),
        pl.BlockSpec((BR, D), lambda i: (i, 0)),
        pl.BlockSpec((BR, D), lambda i: (i, 0)),
        pl.BlockSpec((BR, 1), lambda i: (i, 0)),
        pl.BlockSpec((D, D), lambda i: (0, 0)),
    ],
    out_specs=pl.BlockSpec((BR, D), lambda i: (i, 0)),
    out_shape=jax.ShapeDtypeStruct((N, D), jnp.float32),
)


def kernel(X, edge_index, edge_weight, p, W_conv, W_ih, W_hh, b_ih, b_hh):
    src = edge_index[0]
    dst = edge_index[1]
    deg_parts = _p1()(dst, edge_weight).reshape(NW, N)
    dinv = _dinv(deg_parts)
    pad = EPWP - EPW
    srcw = jnp.pad(src.reshape(NW, EPW), ((0, 0), (0, pad)))
    dstw = jnp.pad(dst.reshape(NW, EPW), ((0, 0), (0, pad)),
                   constant_values=N)
    ww = jnp.pad(edge_weight.reshape(NW, EPW), ((0, 0), (0, pad)))
    packed = srcw | (dstw << 16)
    agg = _p2()(X, packed, ww, dinv)
    w_new = _wnew(X, p, W_conv, W_ih, W_hh, b_ih, b_hh)
    out = _outk(X, agg[0], agg[1], dinv.reshape(N, 1), w_new)
    return out


# speedup vs baseline: 24.4692x; 1.0335x over previous
import functools

import jax
import jax.numpy as jnp
from jax import lax
from jax.experimental import pallas as pl
from jax.experimental.pallas import tpu as pltpu
from jax.experimental.pallas import tpu_sc as plsc

N = 10000
D = 128
E = 320000
K = 128

NC = 2
NS = 16
NW = NC * NS
EPW = E // NW
CH = 48
NCHUNK = 210
EPWP = NCHUNK * CH
NPAD = N + 8
RB = 624
ZCH = 48

_SC_PARAMS = pltpu.CompilerParams(needs_layout_passes=False)


@functools.lru_cache(maxsize=None)
def _mesh():
    return plsc.VectorSubcoreMesh(core_axis_name="c", subcore_axis_name="s")


def _p1_body(dst_hbm, w_hbm, deg_hbm, deg_v, dst_v, w_v):
    c = lax.axis_index("c")
    s = lax.axis_index("s")
    wid = s * NC + c

    ebase = wid * EPW
    pltpu.sync_copy(dst_hbm.at[pl.ds(ebase, EPW)], dst_v)
    pltpu.sync_copy(w_hbm.at[pl.ds(ebase, EPW)], w_v)

    @plsc.parallel_loop(0, N // 16, unroll=8)
    def _(i):
        deg_v[i, pl.ds(0, 16)] = jnp.zeros((16,), jnp.float32)

    @plsc.parallel_loop(0, EPW, step=16, unroll=4)
    def _(j):
        d16 = dst_v[pl.ds(j, 16)]
        w16 = w_v[pl.ds(j, 16)]
        row = lax.shift_right_logical(d16, 4)
        col = lax.bitwise_and(d16, 15)
        plsc.addupdate_scatter(deg_v, [row, col], w16)

    pltpu.sync_copy(deg_v, deg_hbm.at[wid])


@functools.lru_cache(maxsize=None)
def _p1():
    return pl.kernel(
        _p1_body,
        out_type=jax.ShapeDtypeStruct((NW, N // 16, 16), jnp.float32),
        mesh=_mesh(),
        scratch_types=[
            pltpu.VMEM((N // 16, 16), jnp.float32),
            pltpu.VMEM((EPW,), jnp.int32),
            pltpu.VMEM((EPW,), jnp.float32),
        ],
        compiler_params=_SC_PARAMS,
    )


def _dinv_body(parts_ref, dinv_ref):
    dinv_ref[...] = lax.rsqrt(jnp.sum(parts_ref[...], axis=0) + 1.0)


_dinv = pl.pallas_call(
    _dinv_body,
    out_shape=jax.ShapeDtypeStruct((N,), jnp.float32),
)


def _p2_body(x_hbm, pk_hbm, w_hbm, dinv_hbm, agg_hbm,
             pk_v, w_v, s_v, src0, src1, src2, dst0, dst1, dst2,
             rows0, rows1, rows2, dinv_v, acc_sh, gsems, ssems):
    c = lax.axis_index("c")
    s = lax.axis_index("s")
    wid = s * NC + c
    rows = [rows0, rows1, rows2]
    srcb = [src0, src1, src2]
    dstb = [dst0, dst1, dst2]

    pltpu.sync_copy(pk_hbm.at[wid], pk_v)
    pltpu.sync_copy(w_hbm.at[wid], w_v)
    pltpu.sync_copy(dinv_hbm, dinv_v)

    def unpack_idx(ci, b):
        for j in range(0, CH, 16):
            pk16 = pk_v[pl.ds(ci * CH + j, 16)]
            srcb[b][pl.ds(j, 16)] = lax.bitwise_and(pk16, 0xFFFF)
            dstb[b][pl.ds(j, 16)] = lax.shift_right_logical(pk16, 16)

    def issue_gather(ci, b):
        unpack_idx(ci, b)
        pltpu.async_copy(x_hbm.at[srcb[b]], rows[b], gsems.at[b])

    def wait_gather(b):
        pltpu.make_async_copy(x_hbm.at[srcb[b]], rows[b], gsems.at[b]).wait()

    def issue_scatter(b):
        pltpu.async_copy(rows[b], acc_sh.at[dstb[b]], ssems.at[b], add=True)

    def wait_scatter(b):
        pltpu.make_async_copy(rows[b], acc_sh.at[dstb[b]],
                              ssems.at[b]).wait()

    issue_gather(0, 0)
    issue_gather(1, 1)

    @pl.loop(0, ZCH)
    def _(r):
        for cc in range(0, D, 16):
            rows2[r, pl.ds(cc, 16)] = jnp.zeros((16,), jnp.float32)

    rbase = pl.multiple_of(s * RB, 8)

    @pl.loop(0, RB, step=ZCH)
    def _(r0):
        pltpu.sync_copy(rows2, acc_sh.at[pl.ds(rbase + r0, ZCH)])

    @pl.when(s == NS - 1)
    def _():
        pltpu.sync_copy(rows2.at[pl.ds(0, NPAD - NS * RB)],
                        acc_sh.at[pl.ds(NS * RB, NPAD - NS * RB)])

    plsc.subcore_barrier()

    def process(ci, b):
        wait_gather(b)

        for j in range(0, CH, 16):
            idx16 = srcb[b][pl.ds(j, 16)]
            dg = plsc.load_gather(dinv_v, [idx16])
            s_v[pl.ds(j, 16)] = w_v[pl.ds(ci * CH + j, 16)] * dg

        rv = rows[b]

        @plsc.parallel_loop(0, CH, unroll=8)
        def _(j):
            jvec = jnp.full((16,), 0, jnp.int32) + j
            sj = plsc.load_gather(s_v, [jvec])
            for cc in range(0, D, 16):
                rv[j, pl.ds(cc, 16)] = rv[j, pl.ds(cc, 16)] * sj

        issue_scatter(b)

        nxt = ci + 2

        @pl.when(nxt < NCHUNK)
        def _():
            nb = (b + 2) % 3

            @pl.when(nxt >= 3)
            def _():
                wait_scatter(nb)

            issue_gather(nxt, nb)

    @pl.loop(0, NCHUNK // 3)
    def _(ii):
        for u in range(3):
            process(ii * 3 + u, u)

    for b in (0, 1, 2):
        wait_scatter(b)

    plsc.subcore_barrier()

    @pl.loop(0, RB, step=ZCH)
    def _(r0):
        pltpu.sync_copy(acc_sh.at[pl.ds(rbase + r0, ZCH)],
                        agg_hbm.at[c, pl.ds(rbase + r0, ZCH)])

    @pl.when(s == NS - 1)
    def _():
        pltpu.sync_copy(acc_sh.at[pl.ds(NS * RB, 16)],
                        agg_hbm.at[c, pl.ds(NS * RB, 16)])


@functools.lru_cache(maxsize=None)
def _p2():
    return pl.kernel(
        _p2_body,
        out_type=jax.ShapeDtypeStruct((NC, N, D), jnp.float32),
        mesh=_mesh(),
        scratch_types=[
            pltpu.VMEM((EPWP,), jnp.int32),
            pltpu.VMEM((EPWP,), jnp.float32),
            pltpu.VMEM((CH,), jnp.float32),
            pltpu.VMEM((CH,), jnp.int32),
            pltpu.VMEM((CH,), jnp.int32),
            pltpu.VMEM((CH,), jnp.int32),
            pltpu.VMEM((CH,), jnp.int32),
            pltpu.VMEM((CH,), jnp.int32),
            pltpu.VMEM((CH,), jnp.int32),
            pltpu.VMEM((CH, D), jnp.float32),
            pltpu.VMEM((CH, D), jnp.float32),
            pltpu.VMEM((CH, D), jnp.float32),
            pltpu.VMEM((N,), jnp.float32),
            pltpu.VMEM_SHARED((NPAD, D), jnp.float32),
            pltpu.SemaphoreType.DMA((3,)),
            pltpu.SemaphoreType.DMA((3,)),
        ],
        compiler_params=_SC_PARAMS,
    )


def _wnew_body(x_ref, p_ref, wc_ref, wih_ref, whh_ref, bih_ref, bhh_ref,
               wnew_ref, s_scr, xt_scr, idx_s, topv_s):
    p = p_ref[...]
    pn = jnp.sqrt(jnp.sum(p * p))
    s_scr[...] = lax.dot_general(
        x_ref[...].astype(jnp.bfloat16), p.astype(jnp.bfloat16),
        (((1,), (0,)), ((), ())),
        preferred_element_type=jnp.float32) / pn

    lin = lax.broadcasted_iota(jnp.int32, (N,), 0)

    def sel_body(k, carry):
        sv = s_scr[...]
        m = jnp.max(sv)
        idx = jnp.min(jnp.where(sv >= m, lin, jnp.int32(N)))
        idx_s[k] = idx
        topv_s[k] = m
        s_scr[...] = jnp.where(lin == idx, -jnp.inf, sv)
        return carry

    lax.fori_loop(0, K, sel_body, 0)

    def gat_body(k, carry):
        i = idx_s[k]
        g = jnp.tanh(topv_s[k])
        xt_scr[pl.ds(k, 1), :] = x_ref[pl.ds(i, 1), :] * g
        return carry

    lax.fori_loop(0, K, gat_body, 0)

    xt = xt_scr[...]
    h = wc_ref[...]
    dn = (((1,), (1,)), ((), ()))
    gi = lax.dot_general(xt, wih_ref[...], dn, precision=lax.Precision.HIGHEST,
                         preferred_element_type=jnp.float32) + bih_ref[...][None, :]
    gh = lax.dot_general(h, whh_ref[...], dn, precision=lax.Precision.HIGHEST,
                         preferred_element_type=jnp.float32) + bhh_ref[...][None, :]
    r = jax.nn.sigmoid(gi[:, 0:D] + gh[:, 0:D])
    z = jax.nn.sigmoid(gi[:, D:2 * D] + gh[:, D:2 * D])
    n = jnp.tanh(gi[:, 2 * D:3 * D] + r * gh[:, 2 * D:3 * D])
    wnew_ref[...] = (1.0 - z) * n + z * h


_wnew = pl.pallas_call(
    _wnew_body,
    out_shape=jax.ShapeDtypeStruct((D, D), jnp.float32),
    scratch_shapes=[
        pltpu.VMEM((N,), jnp.float32),
        pltpu.VMEM((K, D), jnp.float32),
        pltpu.SMEM((K,), jnp.int32),
        pltpu.SMEM((K,), jnp.float32),
    ],
)


BR = 2000


def _out_body(x_ref, a0_ref, a1_ref, dinv_ref, wn_ref, o_ref):
    dcol = dinv_ref[...]
    t = a0_ref[...] + a1_ref[...] + dcol * x_ref[...]
    t = dcol * t
    o_ref[...] = lax.dot_general(t, wn_ref[...], (((1,), (1,)), ((), ())),
                                 precision=lax.Precision.HIGHEST,
                                 preferred_element_type=jnp.float32)


_outk = pl.pallas_call(
    _out_body,
    grid=(N // BR,),
    in_specs=[
        pl.BlockSpec((BR, D), lambda i: (i, 0)),
        pl.BlockSpec((BR, D), lambda i: (i, 0)),
        pl.BlockSpec((BR, D), lambda i: (i, 0)),
        pl.BlockSpec((BR, 1), lambda i: (i, 0)),
        pl.BlockSpec((D, D), lambda i: (0, 0)),
    ],
    out_specs=pl.BlockSpec((BR, D), lambda i: (i, 0)),
    out_shape=jax.ShapeDtypeStruct((N, D), jnp.float32),
)


def kernel(X, edge_index, edge_weight, p, W_conv, W_ih, W_hh, b_ih, b_hh):
    src = edge_index[0]
    dst = edge_index[1]
    deg_parts = _p1()(dst, edge_weight).reshape(NW, N)
    dinv = _dinv(deg_parts)
    pad = EPWP - EPW
    srcw = jnp.pad(src.reshape(NW, EPW), ((0, 0), (0, pad)))
    dstw = jnp.pad(dst.reshape(NW, EPW), ((0, 0), (0, pad)),
                   constant_values=N)
    ww = jnp.pad(edge_weight.reshape(NW, EPW), ((0, 0), (0, pad)))
    packed = srcw | (dstw << 16)
    agg = _p2()(X, packed, ww, dinv)
    w_new = _wnew(X, p, W_conv, W_ih, W_hh, b_ih, b_hh)
    out = _outk(X, agg[0], agg[1], dinv.reshape(N, 1), w_new)
    return out
